# Initial kernel scaffold; baseline (speedup 1.0000x reference)
import jax
import jax.numpy as jnp
from jax.experimental import pallas as pl


def kernel(positions):
    raise NotImplementedError("write your pallas kernel here")


# speedup vs baseline: 6.2911x; 6.2911x over previous
import math

import jax
import jax.numpy as jnp
from jax import lax
from jax.experimental import pallas as pl

_NUM_NBS = 12
_R0 = 0.5
_D0 = 0.3
_N = 4096
_ROWS = 256

_INTERPRET = False


def _sh6_constants():
    l = 6
    n = []
    for m in range(0, l + 1):
        nlm = math.sqrt(
            (2 * l + 1) / (4.0 * math.pi)
            * math.factorial(l - m) / math.factorial(l + m))
        n.append(nlm if m == 0 else math.sqrt(2.0) * nlm)
    return n


_C = _sh6_constants()


def _stein_block(pb_ref, pt_ref, out_ref):
    i = pl.program_id(0)
    pb = pb_ref[...]
    pt = pt_ref[...]

    ax = pb[:, 0:1]
    ay = pb[:, 1:2]
    az = pb[:, 2:3]
    dx = pt[0:1, :] - ax
    dy = pt[1:2, :] - ay
    dz = pt[2:3, :] - az
    d2 = dx * dx + dy * dy + dz * dz

    col = lax.broadcasted_iota(jnp.int32, (_ROWS, _N), 1)
    work = d2
    mask = jnp.zeros((_ROWS, _N), jnp.float32)
    for k in range(_NUM_NBS + 1):
        m = jnp.min(work, axis=1, keepdims=True)
        eq = work == m
        jmin = jnp.min(jnp.where(eq, col, _N), axis=1, keepdims=True)
        oh = col == jmin
        if k > 0:
            mask = jnp.where(oh, 1.0, mask)
        work = jnp.where(oh, jnp.inf, work)

    d = jnp.sqrt(d2)
    inv = jnp.where(d > 0.0, 1.0 / jnp.where(d > 0.0, d, 1.0), 0.0)
    ux = dx * inv
    uy = dy * inv
    uz = dz * inv

    t = d - _D0
    sig = jnp.exp(t * t * (-1.0 / (2.0 * _R0 * _R0))) * mask
    ssum = jnp.sum(sig, axis=1, keepdims=True)

    z2 = uz * uz
    p = [
        (((231.0 * z2 - 315.0) * z2 + 105.0) * z2 - 5.0) * (1.0 / 16.0),
        ((1386.0 * z2 - 1260.0) * z2 + 210.0) * uz * (1.0 / 16.0),
        ((6930.0 * z2 - 3780.0) * z2 + 210.0) * (1.0 / 16.0),
        (27720.0 * z2 - 7560.0) * uz * (1.0 / 16.0),
        (83160.0 * z2 - 7560.0) * (1.0 / 16.0),
        10395.0 * uz,
    ]

    acc = _C[0] * jnp.sum(sig * p[0], axis=1, keepdims=True)
    norm2 = acc * acc
    a = ux
    b = uy
    for m in range(1, 7):
        pm = p[m] if m < 6 else None
        if m < 6:
            accp = _C[m] * jnp.sum(sig * (pm * a), axis=1, keepdims=True)
            accm = _C[m] * jnp.sum(sig * (pm * b), axis=1, keepdims=True)
        else:
            accp = (_C[6] * 10395.0) * jnp.sum(sig * a, axis=1, keepdims=True)
            accm = (_C[6] * 10395.0) * jnp.sum(sig * b, axis=1, keepdims=True)
        norm2 = norm2 + accp * accp + accm * accm
        if m < 6:
            a, b = a * ux - b * uy, a * uy + b * ux

    qn = jnp.sqrt(norm2) / ssum
    part = jnp.sum(qn, axis=0, keepdims=True) * (1.0 / _N)

    @pl.when(i == 0)
    def _():
        out_ref[...] = jnp.zeros((1, 1), jnp.float32)

    out_ref[...] += part


def kernel(positions):
    pos = positions.astype(jnp.float32)
    pos_t = pos.T
    out = pl.pallas_call(
        _stein_block,
        grid=(_N // _ROWS,),
        in_specs=[
            pl.BlockSpec((_ROWS, 3), lambda i: (i, 0)),
            pl.BlockSpec((3, _N), lambda i: (0, 0)),
        ],
        out_specs=pl.BlockSpec((1, 1), lambda i: (0, 0)),
        out_shape=jax.ShapeDtypeStruct((1, 1), jnp.float32),
        interpret=_INTERPRET,
    )(pos, pos_t)
    return out[0, 0]
